# baseline (speedup 1.0000x reference)
import functools
import math

import jax
import jax.numpy as jnp
import numpy as np
from jax import lax
from jax.experimental import pallas as pl
from jax.experimental.pallas import tpu as pltpu
from jax.experimental.pallas import tpu_sc as plsc

VOCAB = 1000000
D = 64
SEQ = 200
B = 4096
SCALE = math.sqrt(D)
L = 16

NC = 2
NS = 16
NW = NC * NS

CBB = 256
NBQ = B // CBB
CPW = SEQ * NBQ // NW


def _pe_const() -> np.ndarray:
    position = np.arange(SEQ, dtype=np.float32)[:, None]
    num_even = D // 2 + D % 2
    div_term = np.exp(
        np.arange(0, num_even, dtype=np.float32) * (-math.log(10000.0) / D)
    )
    pe = np.zeros((SEQ, D), dtype=np.float32)
    pe[:, 0::2] = np.sin(position * div_term[:num_even])
    pe[:, 1::2] = np.cos(position * div_term[: D // 2])
    return pe


def _sc_embed(xt, table, pe):
    mesh = plsc.VectorSubcoreMesh(
        core_axis_name="c", subcore_axis_name="s", num_cores=NC, num_subcores=NS
    )

    @functools.partial(
        pl.kernel,
        out_type=jax.ShapeDtypeStruct((SEQ, D // 8, B // 128, 8, 128),
                                      jnp.float32),
        mesh=mesh,
        compiler_params=pltpu.CompilerParams(use_tc_tiling_on_sc=False,
                                             needs_layout_passes=False),
        scratch_types=[
            pltpu.VMEM((CBB,), jnp.int32),
            pltpu.VMEM((CBB,), jnp.int32),
            pltpu.VMEM((CBB, D), jnp.float32),
            pltpu.VMEM((CBB, D), jnp.float32),
            pltpu.VMEM((D // 8, CBB // 128, 8, 128), jnp.float32),
            pltpu.VMEM((D // 8, CBB // 128, 8, 128), jnp.float32),
            pltpu.VMEM((SEQ, D), jnp.float32),
            pltpu.SemaphoreType.DMA,
            pltpu.SemaphoreType.DMA,
            pltpu.SemaphoreType.DMA,
            pltpu.SemaphoreType.DMA,
            pltpu.SemaphoreType.DMA,
            pltpu.SemaphoreType.DMA,
        ],
    )
    def k(xt_hbm, table_hbm, pe_hbm, out_hbm,
          idx0, idx1, rows0, rows1, tb0, tb1, pe_v,
          sg0, sg1, sw0, sw1, si0, si1):
        idxs = (idx0, idx1)
        rows = (rows0, rows1)
        tbs = (tb0, tb1)
        sg = (sg0, sg1)
        sw = (sw0, sw1)
        si = (si0, si1)

        wid = lax.axis_index("s") * NC + lax.axis_index("c")
        g0 = wid * CPW
        pltpu.sync_copy(pe_hbm, pe_v)

        def start_idx(b, g):
            s = g // NBQ
            bq = g % NBQ
            pltpu.async_copy(xt_hbm.at[s, pl.ds(bq * CBB, CBB)], idxs[b], si[b])

        def wait_idx(b):
            pltpu.make_async_copy(xt_hbm.at[0, pl.ds(0, CBB)], idxs[b],
                                  si[b]).wait()

        def start_gather(b):
            pltpu.async_copy(table_hbm.at[idxs[b]], rows[b], sg[b])

        def wait_gather(b):
            pltpu.make_async_copy(table_hbm.at[idxs[b]], rows[b], sg[b]).wait()

        def out_slice(g):
            s = g // NBQ
            bq = g % NBQ
            return out_hbm.at[s, pl.ds(0, D // 8), pl.ds(bq * (CBB // 128),
                                                         CBB // 128),
                              pl.ds(0, 8), pl.ds(0, 128)]

        def start_write(b, g):
            pltpu.async_copy(tbs[b], out_slice(g), sw[b])

        def wait_write(b, g):
            pltpu.make_async_copy(tbs[b], out_slice(g), sw[b]).wait()

        start_idx(0, g0)
        start_idx(1, g0 + 1)
        wait_idx(0)
        start_gather(0)

        iota16 = lax.broadcasted_iota(jnp.int32, (L,), 0)

        @pl.loop(0, CPW, step=2)
        def _outer(c0):
            for b in range(2):
                c = c0 + b
                g = g0 + c
                s = g // NBQ

                wait_gather(b)

                @pl.when(c + 2 < CPW)
                def _():
                    start_idx(b, g + 2)

                @pl.when(c + 1 < CPW)
                def _():
                    wait_idx(1 - b)
                    start_gather(1 - b)

                @pl.when(c >= 2)
                def _():
                    wait_write(b, g - 2)

                s_vec = jnp.full((L,), s, jnp.int32)
                rbuf = rows[b]
                tbuf = tbs[b]

                @pl.loop(0, D, unroll=4)
                def _d(d):
                    d_vec = jnp.full((L,), d, jnp.int32)
                    pe_vec = plsc.load_gather(pe_v, [s_vec, d_vec])
                    dB = d // 8
                    di = d % 8
                    for kk in range(CBB // L):
                        ib = iota16 + (kk * L)
                        vals = plsc.load_gather(rbuf, [ib, d_vec])
                        res = vals * SCALE + pe_vec
                        tbuf[dB, kk // 8, di, pl.ds((kk % 8) * L, L)] = res

                start_write(b, g)

        wait_write(0, g0 + CPW - 2)
        wait_write(1, g0 + CPW - 1)

    return k(xt, table, pe)


def kernel(x, table):
    xt = x.astype(jnp.int32).T
    pe = jnp.asarray(_pe_const())
    out5 = _sc_embed(xt, table, pe)
    return jnp.transpose(out5, (2, 4, 0, 1, 3)).reshape(B, SEQ, D)


# speedup vs baseline: 1.6181x; 1.6181x over previous
import functools
import math

import jax
import jax.numpy as jnp
import numpy as np
from jax import lax
from jax.experimental import pallas as pl
from jax.experimental.pallas import tpu as pltpu
from jax.experimental.pallas import tpu_sc as plsc

VOCAB = 1000000
D = 64
SEQ = 200
B = 4096
SCALE = math.sqrt(D)
L = 16

NC = 2
NS = 16
NW = NC * NS

CBB = 256
NBQ = B // CBB
CPW = SEQ * NBQ // NW


def _pe_const() -> np.ndarray:
    position = np.arange(SEQ, dtype=np.float32)[:, None]
    num_even = D // 2 + D % 2
    div_term = np.exp(
        np.arange(0, num_even, dtype=np.float32) * (-math.log(10000.0) / D)
    )
    pe = np.zeros((SEQ, D), dtype=np.float32)
    pe[:, 0::2] = np.sin(position * div_term[:num_even])
    pe[:, 1::2] = np.cos(position * div_term[: D // 2])
    return pe


def _sc_embed(xt, table, pe):
    mesh = plsc.VectorSubcoreMesh(
        core_axis_name="c", subcore_axis_name="s", num_cores=NC, num_subcores=NS
    )

    @functools.partial(
        pl.kernel,
        out_type=jax.ShapeDtypeStruct((SEQ, D // 8, B // 128, 8, 128),
                                      jnp.float32),
        mesh=mesh,
        compiler_params=pltpu.CompilerParams(use_tc_tiling_on_sc=False,
                                             needs_layout_passes=False),
        scratch_types=[
            pltpu.VMEM((CBB,), jnp.int32),
            pltpu.VMEM((CBB,), jnp.int32),
            pltpu.VMEM((CBB, D), jnp.float32),
            pltpu.VMEM((CBB, D), jnp.float32),
            pltpu.VMEM((D // 8, CBB // 128, 8, 128), jnp.float32),
            pltpu.VMEM((D // 8, CBB // 128, 8, 128), jnp.float32),
            pltpu.VMEM((SEQ, D), jnp.float32),
            pltpu.SemaphoreType.DMA,
            pltpu.SemaphoreType.DMA,
            pltpu.SemaphoreType.DMA,
            pltpu.SemaphoreType.DMA,
            pltpu.SemaphoreType.DMA,
            pltpu.SemaphoreType.DMA,
        ],
    )
    def k(xt_hbm, table_hbm, pe_hbm, out_hbm,
          idx0, idx1, rows0, rows1, tb0, tb1, pe_v,
          sg0, sg1, sw0, sw1, si0, si1):
        idxs = (idx0, idx1)
        rows = (rows0, rows1)
        tbs = (tb0, tb1)
        sg = (sg0, sg1)
        sw = (sw0, sw1)
        si = (si0, si1)

        wid = lax.axis_index("s") * NC + lax.axis_index("c")
        g0 = wid * CPW
        pltpu.sync_copy(pe_hbm, pe_v)

        def start_idx(b, g):
            s = g // NBQ
            bq = g % NBQ
            pltpu.async_copy(xt_hbm.at[s, pl.ds(bq * CBB, CBB)], idxs[b], si[b])

        def wait_idx(b):
            pltpu.make_async_copy(xt_hbm.at[0, pl.ds(0, CBB)], idxs[b],
                                  si[b]).wait()

        def start_gather(b):
            pltpu.async_copy(table_hbm.at[idxs[b]], rows[b], sg[b])

        def wait_gather(b):
            pltpu.make_async_copy(table_hbm.at[idxs[b]], rows[b], sg[b]).wait()

        def out_slice(g):
            s = g // NBQ
            bq = g % NBQ
            return out_hbm.at[s, pl.ds(0, D // 8), pl.ds(bq * (CBB // 128),
                                                         CBB // 128),
                              pl.ds(0, 8), pl.ds(0, 128)]

        def start_write(b, g):
            pltpu.async_copy(tbs[b], out_slice(g), sw[b])

        def wait_write(b, g):
            pltpu.make_async_copy(tbs[b], out_slice(g), sw[b]).wait()

        start_idx(0, g0)
        start_idx(1, g0 + 1)
        wait_idx(0)
        start_gather(0)

        iota16 = lax.broadcasted_iota(jnp.int32, (L,), 0)

        @pl.loop(0, CPW, step=2)
        def _outer(c0):
            for b in range(2):
                c = c0 + b
                g = g0 + c
                s = g // NBQ

                wait_gather(b)

                @pl.when(c + 2 < CPW)
                def _():
                    start_idx(b, g + 2)

                @pl.when(c + 1 < CPW)
                def _():
                    wait_idx(1 - b)
                    start_gather(1 - b)

                @pl.when(c >= 2)
                def _():
                    wait_write(b, g - 2)

                s_vec = jnp.full((L,), s, jnp.int32)
                rbuf = rows[b]
                tbuf = tbs[b]

                @plsc.parallel_loop(0, D, unroll=4)
                def _d(d):
                    d_vec = jnp.full((L,), d, jnp.int32)
                    pe_vec = plsc.load_gather(pe_v, [s_vec, d_vec])
                    dB = d // 8
                    di = d % 8
                    for kk in range(CBB // L):
                        ib = iota16 + (kk * L)
                        vals = plsc.load_gather(rbuf, [ib, d_vec])
                        res = vals * SCALE + pe_vec
                        tbuf[dB, kk // 8, di, pl.ds((kk % 8) * L, L)] = res

                start_write(b, g)

        wait_write(0, g0 + CPW - 2)
        wait_write(1, g0 + CPW - 1)

    return k(xt, table, pe)


def kernel(x, table):
    xt = x.astype(jnp.int32).T
    pe = jnp.asarray(_pe_const())
    out5 = _sc_embed(xt, table, pe)
    return jnp.transpose(out5, (2, 4, 0, 1, 3)).reshape(B, SEQ, D)
